# baseline (speedup 1.0000x reference)
import functools

import jax
import jax.numpy as jnp
import numpy as np
from jax.experimental import pallas as pl
from jax.experimental.pallas import tpu as pltpu

_RC = 0.25
_S = 27
_CENTER = _S // 2


def _shift_grid():
    r = np.array([-1, 0, 1])
    g = np.stack(np.meshgrid(r, r, r, indexing="ij"), axis=-1).reshape(-1, 3)
    return jnp.asarray(g, dtype=jnp.float32)


def _plane_kernel(row_ref, col_ref, vec_ref, sod_ref, *, n):
    s = pl.program_id(1)
    ii = jax.lax.broadcasted_iota(jnp.int32, (n, 1), 0)
    jj = jax.lax.broadcasted_iota(jnp.int32, (1, n), 1)

    dx = row_ref[0:1, :] - col_ref[:, 0:1]
    dy = row_ref[1:2, :] - col_ref[:, 1:2]
    dz = row_ref[2:3, :] - col_ref[:, 2:3]
    sod = dx * dx + dy * dy + dz * dz
    mask = (sod < _RC * _RC) & jnp.logical_or(ii != jj, s != _CENTER)

    zero = jnp.float32(0.0)
    vec_ref[0] = jnp.where(mask, dx, zero)
    vec_ref[1] = jnp.where(mask, dy, zero)
    vec_ref[2] = jnp.where(mask, dz, zero)
    sod_ref[...] = jnp.where(mask, sod, zero)


@jax.jit
def kernel(pos, cel, pbc, ent):
    B, N, _ = pos.shape
    f32 = jnp.float32
    sft = _shift_grid()
    sft_xyz = jnp.einsum("sc,bcd->bsd", sft, cel)
    valid = jnp.all(pbc[:, None, :] | (sft[None, :, :] == 0), axis=-1)

    s_off = (65536.0 * (jnp.arange(_S, dtype=f32) + 1.0))[None, :, None]
    sft_eff = sft_xyz + jnp.where(valid[..., None], 0.0, s_off)
    entf = (~ent).astype(f32)[..., None]
    posj_eff = pos + entf * 4096.0
    posi_eff = pos - entf * 16777216.0

    row_eff = (
        posj_eff.transpose(0, 2, 1)[:, None, :, :] + sft_eff[..., None]
    )

    grid = (B, _S)
    vec_t, sod_t = pl.pallas_call(
        functools.partial(_plane_kernel, n=N),
        grid=grid,
        in_specs=[
            pl.BlockSpec((None, None, 3, N), lambda b, s: (b, s, 0, 0)),
            pl.BlockSpec((None, N, 3), lambda b, s: (b, 0, 0)),
        ],
        out_specs=[
            pl.BlockSpec((None, None, 3, N, N), lambda b, s: (b, s, 0, 0, 0)),
            pl.BlockSpec((None, None, N, N), lambda b, s: (b, s, 0, 0)),
        ],
        out_shape=[
            jax.ShapeDtypeStruct((B, _S, 3, N, N), f32),
            jax.ShapeDtypeStruct((B, _S, N, N), f32),
        ],
        compiler_params=pltpu.CompilerParams(
            dimension_semantics=("parallel", "arbitrary"),
        ),
    )(row_eff, posi_eff)

    vec = jnp.transpose(vec_t, (0, 3, 4, 1, 2))
    sod = jnp.transpose(sod_t, (0, 2, 3, 1))
    return vec, sod


# speedup vs baseline: 1.0024x; 1.0024x over previous
import functools

import jax
import jax.numpy as jnp
import numpy as np
from jax.experimental import pallas as pl

_RC = 0.25
_S = 27
_CENTER = _S // 2


def _shift_grid():
    r = np.array([-1, 0, 1])
    g = np.stack(np.meshgrid(r, r, r, indexing="ij"), axis=-1).reshape(-1, 3)
    return jnp.asarray(g, dtype=jnp.float32)


def _plane_kernel(row_ref, col_ref, vec_ref, sod_ref, *, n):
    s = pl.program_id(1)
    ii = jax.lax.broadcasted_iota(jnp.int32, (n, 1), 0)
    jj = jax.lax.broadcasted_iota(jnp.int32, (1, n), 1)

    dx = row_ref[0:1, :] - col_ref[:, 0:1]
    dy = row_ref[1:2, :] - col_ref[:, 1:2]
    dz = row_ref[2:3, :] - col_ref[:, 2:3]
    sod = dx * dx + dy * dy + dz * dz
    mask = (sod < _RC * _RC) & jnp.logical_or(ii != jj, s != _CENTER)

    zero = jnp.float32(0.0)
    vec_ref[0] = jnp.where(mask, dx, zero)
    vec_ref[1] = jnp.where(mask, dy, zero)
    vec_ref[2] = jnp.where(mask, dz, zero)
    sod_ref[...] = jnp.where(mask, sod, zero)


def _planes(row_eff, posi_eff, n):
    b = row_eff.shape[0]
    return pl.pallas_call(
        functools.partial(_plane_kernel, n=n),
        grid=(b, _S),
        in_specs=[
            pl.BlockSpec((None, None, 3, n), lambda bb, s: (bb, s, 0, 0)),
            pl.BlockSpec((None, n, 3), lambda bb, s: (bb, 0, 0)),
        ],
        out_specs=[
            pl.BlockSpec((None, None, 3, n, n), lambda bb, s: (bb, s, 0, 0, 0)),
            pl.BlockSpec((None, None, n, n), lambda bb, s: (bb, s, 0, 0)),
        ],
        out_shape=[
            jax.ShapeDtypeStruct((b, _S, 3, n, n), jnp.float32),
            jax.ShapeDtypeStruct((b, _S, n, n), jnp.float32),
        ],
    )(row_eff, posi_eff)


@jax.jit
def kernel(pos, cel, pbc, ent):
    B, N, _ = pos.shape
    f32 = jnp.float32
    sft = _shift_grid()
    sft_xyz = jnp.einsum("sc,bcd->bsd", sft, cel)
    valid = jnp.all(pbc[:, None, :] | (sft[None, :, :] == 0), axis=-1)

    s_off = (65536.0 * (jnp.arange(_S, dtype=f32) + 1.0))[None, :, None]
    sft_eff = sft_xyz + jnp.where(valid[..., None], 0.0, s_off)
    entf = (~ent).astype(f32)[..., None]
    posj_eff = pos + entf * 4096.0
    posi_eff = pos - entf * 16777216.0

    row_eff = (
        posj_eff.transpose(0, 2, 1)[:, None, :, :] + sft_eff[..., None]
    )

    vec_t, sod_t = _planes(row_eff, posi_eff, N)

    vec = jnp.transpose(vec_t, (0, 3, 4, 1, 2))
    sod = jnp.transpose(sod_t, (0, 2, 3, 1))
    return vec, sod


# speedup vs baseline: 1.0340x; 1.0315x over previous
import functools

import jax
import jax.numpy as jnp
import numpy as np
from jax.experimental import pallas as pl

_RC = 0.25
_S = 27
_CENTER = _S // 2


def _shift_grid():
    r = np.array([-1, 0, 1])
    g = np.stack(np.meshgrid(r, r, r, indexing="ij"), axis=-1).reshape(-1, 3)
    return jnp.asarray(g, dtype=jnp.float32)


def _plane_kernel(row_ref, col_ref, vec_ref, sod_ref, *, n, sb):
    s0 = pl.program_id(1) * sb
    ii = jax.lax.broadcasted_iota(jnp.int32, (n, 1), 0)
    jj = jax.lax.broadcasted_iota(jnp.int32, (1, n), 1)
    zero = jnp.float32(0.0)

    for k in range(sb):
        s = s0 + k
        dx = row_ref[k, 0:1, :] - col_ref[:, 0:1]
        dy = row_ref[k, 1:2, :] - col_ref[:, 1:2]
        dz = row_ref[k, 2:3, :] - col_ref[:, 2:3]
        sod = dx * dx + dy * dy + dz * dz
        mask = (sod < _RC * _RC) & jnp.logical_or(ii != jj, s != _CENTER)

        vec_ref[k, 0] = jnp.where(mask, dx, zero)
        vec_ref[k, 1] = jnp.where(mask, dy, zero)
        vec_ref[k, 2] = jnp.where(mask, dz, zero)
        sod_ref[k] = jnp.where(mask, sod, zero)


def _planes(row_eff, posi_eff, n):
    b = row_eff.shape[0]
    sb = 3
    return pl.pallas_call(
        functools.partial(_plane_kernel, n=n, sb=sb),
        grid=(b, _S // sb),
        in_specs=[
            pl.BlockSpec((None, sb, 3, n), lambda bb, s: (bb, s, 0, 0)),
            pl.BlockSpec((None, n, 3), lambda bb, s: (bb, 0, 0)),
        ],
        out_specs=[
            pl.BlockSpec((None, sb, 3, n, n), lambda bb, s: (bb, s, 0, 0, 0)),
            pl.BlockSpec((None, sb, n, n), lambda bb, s: (bb, s, 0, 0)),
        ],
        out_shape=[
            jax.ShapeDtypeStruct((b, _S, 3, n, n), jnp.float32),
            jax.ShapeDtypeStruct((b, _S, n, n), jnp.float32),
        ],
    )(row_eff, posi_eff)


@jax.jit
def kernel(pos, cel, pbc, ent):
    B, N, _ = pos.shape
    f32 = jnp.float32
    sft = _shift_grid()
    sft_xyz = jnp.einsum("sc,bcd->bsd", sft, cel)
    valid = jnp.all(pbc[:, None, :] | (sft[None, :, :] == 0), axis=-1)

    s_off = (65536.0 * (jnp.arange(_S, dtype=f32) + 1.0))[None, :, None]
    sft_eff = sft_xyz + jnp.where(valid[..., None], 0.0, s_off)
    entf = (~ent).astype(f32)[..., None]
    posj_eff = pos + entf * 4096.0
    posi_eff = pos - entf * 16777216.0

    row_eff = (
        posj_eff.transpose(0, 2, 1)[:, None, :, :] + sft_eff[..., None]
    )

    vec_t, sod_t = _planes(row_eff, posi_eff, N)

    vec = jnp.transpose(vec_t, (0, 3, 4, 1, 2))
    sod = jnp.transpose(sod_t, (0, 2, 3, 1))
    return vec, sod


# speedup vs baseline: 1.0475x; 1.0131x over previous
import functools

import jax
import jax.numpy as jnp
import numpy as np
from jax.experimental import pallas as pl

_RC = 0.25
_S = 27
_CENTER = _S // 2


def _shift_grid():
    r = np.array([-1, 0, 1])
    g = np.stack(np.meshgrid(r, r, r, indexing="ij"), axis=-1).reshape(-1, 3)
    return jnp.asarray(g, dtype=jnp.float32)


def _plane_kernel(row_ref, col_ref, vec_ref, sod_ref, *, n, sb):
    zero = jnp.float32(0.0)
    for k in range(sb):
        dx = row_ref[k, 0:1, :] - col_ref[:, 0:1]
        dy = row_ref[k, 1:2, :] - col_ref[:, 1:2]
        dz = row_ref[k, 2:3, :] - col_ref[:, 2:3]
        sod = dx * dx + dy * dy + dz * dz
        mask = sod < _RC * _RC

        vec_ref[k, 0] = jnp.where(mask, dx, zero)
        vec_ref[k, 1] = jnp.where(mask, dy, zero)
        vec_ref[k, 2] = jnp.where(mask, dz, zero)
        sod_ref[k] = jnp.where(mask, sod, zero)


def _planes(row_eff, posi_eff, n):
    b = row_eff.shape[0]
    sb = 3
    return pl.pallas_call(
        functools.partial(_plane_kernel, n=n, sb=sb),
        grid=(b, _S // sb),
        in_specs=[
            pl.BlockSpec((None, sb, 3, n), lambda bb, s: (bb, s, 0, 0)),
            pl.BlockSpec((None, n, 3), lambda bb, s: (bb, 0, 0)),
        ],
        out_specs=[
            pl.BlockSpec((None, sb, 3, n, n), lambda bb, s: (bb, s, 0, 0, 0)),
            pl.BlockSpec((None, sb, n, n), lambda bb, s: (bb, s, 0, 0)),
        ],
        out_shape=[
            jax.ShapeDtypeStruct((b, _S, 3, n, n), jnp.float32),
            jax.ShapeDtypeStruct((b, _S, n, n), jnp.float32),
        ],
    )(row_eff, posi_eff)


@jax.jit
def kernel(pos, cel, pbc, ent):
    B, N, _ = pos.shape
    f32 = jnp.float32
    sft = _shift_grid()
    sft_xyz = jnp.einsum("sc,bcd->bsd", sft, cel)
    valid = jnp.all(pbc[:, None, :] | (sft[None, :, :] == 0), axis=-1)

    s_off = (65536.0 * (jnp.arange(_S, dtype=f32) + 1.0))[None, :, None]
    sft_eff = sft_xyz + jnp.where(valid[..., None], 0.0, s_off)
    entf = (~ent).astype(f32)[..., None]
    posj_eff = pos + entf * 4096.0
    posi_eff = pos - entf * 16777216.0

    row_eff = (
        posj_eff.transpose(0, 2, 1)[:, None, :, :] + sft_eff[..., None]
    )

    vec_t, sod_t = _planes(row_eff, posi_eff, N)

    vec = jnp.transpose(vec_t, (0, 3, 4, 1, 2))
    sod = jnp.transpose(sod_t, (0, 2, 3, 1))
    return vec, sod
